# Initial kernel scaffold; baseline (speedup 1.0000x reference)
import jax
import jax.numpy as jnp
from jax.experimental import pallas as pl


def kernel(inputs):
    raise NotImplementedError("write your pallas kernel here")


# speedup vs baseline: 3.9225x; 3.9225x over previous
import functools

import numpy as np
import jax
import jax.numpy as jnp
from jax import lax
from jax.experimental import pallas as pl
from jax.experimental.pallas import tpu as pltpu

_K = 128
_LANES = 128


def _roll_up(v, d, n):
    return jnp.concatenate([v[d:], v[:d]], axis=0)


def _roll_dn(v, d, n):
    return jnp.concatenate([v[n - d:], v[:n - d]], axis=0)


def _ce(v, k, d, flip, n):
    i = lax.broadcasted_iota(jnp.int32, (n, 1), 0)
    ld = d.bit_length() - 1
    fm = ((i >> ld) & 1) == 0
    desc = ((i >> k) & 1) == 0
    want_max = jnp.logical_xor(fm == desc, flip)
    up = _roll_up(v, d, n)
    dn = _roll_dn(v, d, n)
    partner = jnp.where(fm, up, dn)
    return jnp.where(want_max, jnp.maximum(v, partner), jnp.minimum(v, partner))


def _bitonic_sort(v, flip, n):
    lg = n.bit_length() - 1
    for k in range(1, lg + 1):
        d = 1 << (k - 1)
        while d >= 1:
            v = _ce(v, k, d, flip, n)
            d >>= 1
    return v


def _bitonic_merge(v, flip, n):
    lg = n.bit_length() - 1
    d = n >> 1
    while d >= 1:
        v = _ce(v, lg, d, flip, n)
        d >>= 1
    return v


def _body(x_ref, o_ref, s_ref, *, s, k):
    nchunks = s // k

    def p1(c, carry):
        base = pl.multiple_of(c * k, k)
        v = x_ref[0, pl.ds(base, k), :]
        flip = (c % 2) == 1
        v = _bitonic_sort(v, flip, k)
        s_ref[pl.ds(base, k), :] = v
        return carry

    lax.fori_loop(0, nchunks, p1, 0)

    lists = nchunks
    while lists > 1:
        def p2(j, carry):
            a_base = pl.multiple_of(j * (2 * k), k)
            b_base = pl.multiple_of(j * (2 * k) + k, k)
            o_base = pl.multiple_of(j * k, k)
            a = s_ref[pl.ds(a_base, k), :]
            b = s_ref[pl.ds(b_base, k), :]
            m = jnp.maximum(a, b)
            flip = (j % 2) == 1
            m = _bitonic_merge(m, flip, k)
            s_ref[pl.ds(o_base, k), :] = m
            return carry

        lax.fori_loop(0, lists // 2, p2, 0)
        lists //= 2

    o_ref[0] = s_ref[0:k, :]


def kernel(inputs):
    b, s, d = inputs.shape
    grid = (b, d // _LANES)
    out = pl.pallas_call(
        functools.partial(_body, s=s, k=_K),
        grid=grid,
        in_specs=[pl.BlockSpec((1, s, _LANES), lambda bi, j: (bi, 0, j))],
        out_specs=pl.BlockSpec((1, _K, _LANES), lambda bi, j: (bi, 0, j)),
        out_shape=jax.ShapeDtypeStruct((b, _K, d), jnp.float32),
        scratch_shapes=[pltpu.VMEM((s, _LANES), jnp.float32)],
    )(inputs)
    return out


# speedup vs baseline: 26.3472x; 6.7170x over previous
import functools

import jax
import jax.numpy as jnp
from jax import lax
from jax.experimental import pallas as pl
from jax.experimental.pallas import tpu as pltpu

_K = 128
_LANES = 128
_G = 1024


def _ce_sort(v, k, d, asc):
    lanes = v.shape[-1]
    dd = 8 * d
    r_pairs = _G // (2 * dd)
    x = v.reshape(r_pairs, 2, dd, lanes)
    a, b = x[:, 0], x[:, 1]
    mx = jnp.maximum(a, b)
    mn = jnp.minimum(a, b)
    nblk = 64 >> k
    if nblk == 0:
        top, bot = (mn, mx) if asc else (mx, mn)
    else:
        p = (1 << (k - 1)) // d
        mx5 = mx.reshape(nblk, 2, p, dd, lanes)
        mn5 = mn.reshape(nblk, 2, p, dd, lanes)
        if asc:
            mx5, mn5 = mn5, mx5
        top = jnp.concatenate([mx5[:, 0:1], mn5[:, 1:2]], axis=1)
        bot = jnp.concatenate([mn5[:, 0:1], mx5[:, 1:2]], axis=1)
        top = top.reshape(r_pairs, dd, lanes)
        bot = bot.reshape(r_pairs, dd, lanes)
    return jnp.stack([top, bot], axis=1).reshape(_G, lanes)


def _ce_clean(v, d, asc, rmask):
    lanes = v.shape[-1]
    dd = 8 * d
    r_pairs = _G // (2 * dd)
    x = v.reshape(r_pairs, 2, dd, lanes)
    a, b = x[:, 0], x[:, 1]
    mx = jnp.maximum(a, b)
    mn = jnp.minimum(a, b)
    if rmask is None:
        top, bot = (mn, mx) if asc else (mx, mn)
    else:
        rm = rmask.reshape(r_pairs, 2, dd, 1)[:, 0]
        top = jnp.where(rm, mx, mn)
        bot = jnp.where(rm, mn, mx)
    return jnp.stack([top, bot], axis=1).reshape(_G, lanes)


def _sort_group(v, asc):
    for k in range(1, 8):
        d = 1 << (k - 1)
        while d:
            v = _ce_sort(v, k, d, asc)
            d >>= 1
    return v


def _cleanup(v, asc, rmask):
    for d in (64, 32, 16, 8, 4, 2, 1):
        v = _ce_clean(v, d, asc, rmask)
    return v


def _rmask(shift):
    i = lax.broadcasted_iota(jnp.int32, (_G, 1), 0)
    return (((i & 7) >> shift) & 1) == 0


def _body(x_ref, o_ref, s_ref, *, s):
    ngroups = s // _G

    def p1(t, carry):
        for off, asc in ((0, False), (1, True)):
            base = pl.multiple_of((2 * t + off) * _G, _G)
            v = x_ref[0, pl.ds(base, _G), :]
            s_ref[pl.ds(base, _G), :] = _sort_group(v, asc)
        return carry

    lax.fori_loop(0, ngroups // 2, p1, 0)

    def merge_groups(ga, gb, asc, rmask):
        a = s_ref[pl.ds(pl.multiple_of(ga * _G, _G), _G), :]
        b = s_ref[pl.ds(pl.multiple_of(gb * _G, _G), _G), :]
        m = _cleanup(jnp.maximum(a, b), asc, rmask)
        s_ref[pl.ds(pl.multiple_of(ga * _G, _G), _G), :] = m

    def l1(u, carry):
        merge_groups(4 * u, 4 * u + 1, False, None)
        merge_groups(4 * u + 2, 4 * u + 3, True, None)
        return carry

    lax.fori_loop(0, ngroups // 4, l1, 0)
    merge_groups(0, 2, False, None)
    merge_groups(4, 6, True, None)
    merge_groups(0, 4, False, _rmask(0))

    v = s_ref[0:_G, :]
    w = jnp.concatenate([v[1:], v[:1]], axis=0)
    v = _cleanup(jnp.maximum(v, w), False, _rmask(1))
    w = jnp.concatenate([v[2:], v[:2]], axis=0)
    v = _cleanup(jnp.maximum(v, w), False, _rmask(2))
    w = jnp.concatenate([v[4:], v[:4]], axis=0)
    v = _cleanup(jnp.maximum(v, w), False, None)

    o_ref[0] = v.reshape(_K, 8, v.shape[-1])[:, 0, :]


def kernel(inputs):
    b, s, d = inputs.shape
    grid = (b, d // _LANES)
    out = pl.pallas_call(
        functools.partial(_body, s=s),
        grid=grid,
        in_specs=[pl.BlockSpec((1, s, _LANES), lambda bi, j: (bi, 0, j))],
        out_specs=pl.BlockSpec((1, _K, _LANES), lambda bi, j: (bi, 0, j)),
        out_shape=jax.ShapeDtypeStruct((b, _K, d), jnp.float32),
        scratch_shapes=[pltpu.VMEM((s, _LANES), jnp.float32)],
    )(inputs)
    return out


# speedup vs baseline: 35.7521x; 1.3570x over previous
import functools

import jax
import jax.numpy as jnp
from jax import lax
from jax.experimental import pallas as pl
from jax.experimental.pallas import tpu as pltpu

_K = 128
_LANES = 128
_G = 1024


def _ce_sort(v, k, d, asc, row0=0):
    n, lanes = v.shape
    dd = 8 * d
    r_pairs = n // (2 * dd)
    x = v.reshape(r_pairs, 2, dd, lanes)
    a, b = x[:, 0], x[:, 1]
    mx = jnp.maximum(a, b)
    mn = jnp.minimum(a, b)
    nblk = n >> (k + 4)
    if nblk == 0:
        if (row0 >> (k + 3)) & 1:
            asc = not asc
        top, bot = (mn, mx) if asc else (mx, mn)
    else:
        p = (1 << (k - 1)) // d
        mx5 = mx.reshape(nblk, 2, p, dd, lanes)
        mn5 = mn.reshape(nblk, 2, p, dd, lanes)
        if asc:
            mx5, mn5 = mn5, mx5
        top = jnp.concatenate([mx5[:, 0:1], mn5[:, 1:2]], axis=1)
        bot = jnp.concatenate([mn5[:, 0:1], mx5[:, 1:2]], axis=1)
        top = top.reshape(r_pairs, dd, lanes)
        bot = bot.reshape(r_pairs, dd, lanes)
    return jnp.stack([top, bot], axis=1).reshape(n, lanes)


def _ce_clean(v, d, asc, rmask):
    n, lanes = v.shape
    dd = 8 * d
    r_pairs = n // (2 * dd)
    x = v.reshape(r_pairs, 2, dd, lanes)
    a, b = x[:, 0], x[:, 1]
    mx = jnp.maximum(a, b)
    mn = jnp.minimum(a, b)
    if rmask is None:
        top, bot = (mn, mx) if asc else (mx, mn)
    else:
        rm = rmask.reshape(r_pairs, 2, dd, 1)[:, 0]
        top = jnp.where(rm, mx, mn)
        bot = jnp.where(rm, mn, mx)
    return jnp.stack([top, bot], axis=1).reshape(n, lanes)


def _sort_group(v, asc):
    for k in range(1, 8):
        d = 1 << (k - 1)
        while d >= 16:
            v = _ce_sort(v, k, d, asc)
            d >>= 1
        segs = []
        for si in range(v.shape[0] // 128):
            w = v[si * 128:(si + 1) * 128]
            ds_ = min(1 << (k - 1), 8)
            while ds_:
                w = _ce_sort(w, k, ds_, asc, row0=si * 128)
                ds_ >>= 1
            segs.append(w)
        v = jnp.concatenate(segs, axis=0)
    return v


def _cleanup(v, asc, rmask):
    for d in (64, 32, 16):
        v = _ce_clean(v, d, asc, rmask)
    segs = []
    for si in range(v.shape[0] // 128):
        w = v[si * 128:(si + 1) * 128]
        rm = None if rmask is None else rmask[si * 128:(si + 1) * 128]
        for d in (8, 4, 2, 1):
            w = _ce_clean(w, d, asc, rm)
        segs.append(w)
    return jnp.concatenate(segs, axis=0)


def _rmask(shift):
    i = lax.broadcasted_iota(jnp.int32, (_G, 1), 0)
    return (((i & 7) >> shift) & 1) == 0


def _body(x_ref, o_ref, s_ref, *, s):
    ngroups = s // _G

    def p1(t, carry):
        for off, asc in ((0, False), (1, True)):
            base = pl.multiple_of((2 * t + off) * _G, _G)
            v = x_ref[0, pl.ds(base, _G), :]
            s_ref[pl.ds(base, _G), :] = _sort_group(v, asc)
        return carry

    lax.fori_loop(0, ngroups // 2, p1, 0)

    def merge_groups(ga, gb, asc, rmask):
        a = s_ref[pl.ds(pl.multiple_of(ga * _G, _G), _G), :]
        b = s_ref[pl.ds(pl.multiple_of(gb * _G, _G), _G), :]
        m = _cleanup(jnp.maximum(a, b), asc, rmask)
        s_ref[pl.ds(pl.multiple_of(ga * _G, _G), _G), :] = m

    def l1(u, carry):
        merge_groups(4 * u, 4 * u + 1, False, None)
        merge_groups(4 * u + 2, 4 * u + 3, True, None)
        return carry

    lax.fori_loop(0, ngroups // 4, l1, 0)
    merge_groups(0, 2, False, None)
    merge_groups(4, 6, True, None)
    merge_groups(0, 4, False, _rmask(0))

    v = s_ref[0:_G, :]
    w = jnp.concatenate([v[1:], v[:1]], axis=0)
    v = _cleanup(jnp.maximum(v, w), False, _rmask(1))
    w = jnp.concatenate([v[2:], v[:2]], axis=0)
    v = _cleanup(jnp.maximum(v, w), False, _rmask(2))
    w = jnp.concatenate([v[4:], v[:4]], axis=0)
    v = _cleanup(jnp.maximum(v, w), False, None)

    o_ref[0] = v.reshape(_K, 8, v.shape[-1])[:, 0, :]


def kernel(inputs):
    b, s, d = inputs.shape
    grid = (b, d // _LANES)
    out = pl.pallas_call(
        functools.partial(_body, s=s),
        grid=grid,
        in_specs=[pl.BlockSpec((1, s, _LANES), lambda bi, j: (bi, 0, j))],
        out_specs=pl.BlockSpec((1, _K, _LANES), lambda bi, j: (bi, 0, j)),
        out_shape=jax.ShapeDtypeStruct((b, _K, d), jnp.float32),
        scratch_shapes=[pltpu.VMEM((s, _LANES), jnp.float32)],
    )(inputs)
    return out


# speedup vs baseline: 39.8029x; 1.1133x over previous
import functools

import jax
import jax.numpy as jnp
from jax import lax
from jax.experimental import pallas as pl
from jax.experimental.pallas import tpu as pltpu

_K = 128
_LANES = 128
_G = 1024


def _ce_sort(v, k, d, row0=0):
    n, lanes = v.shape
    dd = 8 * d
    r_pairs = n // (2 * dd)
    x = v.reshape(r_pairs, 2, dd, lanes)
    a, b = x[:, 0], x[:, 1]
    mx = jnp.maximum(a, b)
    mn = jnp.minimum(a, b)
    nblk = n >> (k + 4)
    if nblk == 0:
        if (row0 >> (k + 3)) & 1:
            top, bot = mn, mx
        else:
            top, bot = mx, mn
    else:
        p = (1 << (k - 1)) // d
        mx5 = mx.reshape(nblk, 2, p, dd, lanes)
        mn5 = mn.reshape(nblk, 2, p, dd, lanes)
        top = jnp.concatenate([mx5[:, 0:1], mn5[:, 1:2]], axis=1)
        bot = jnp.concatenate([mn5[:, 0:1], mx5[:, 1:2]], axis=1)
        top = top.reshape(r_pairs, dd, lanes)
        bot = bot.reshape(r_pairs, dd, lanes)
    return jnp.stack([top, bot], axis=1).reshape(n, lanes)


def _ce_clean(v, d):
    n, lanes = v.shape
    dd = 8 * d
    r_pairs = n // (2 * dd)
    x = v.reshape(r_pairs, 2, dd, lanes)
    a, b = x[:, 0], x[:, 1]
    mx = jnp.maximum(a, b)
    mn = jnp.minimum(a, b)
    return jnp.stack([mx, mn], axis=1).reshape(n, lanes)


def _sort_group(v):
    for k in range(1, 8):
        d = 1 << (k - 1)
        while d >= 16:
            v = _ce_sort(v, k, d)
            d >>= 1
        segs = []
        for si in range(v.shape[0] // 128):
            w = v[si * 128:(si + 1) * 128]
            ds_ = min(1 << (k - 1), 8)
            while ds_:
                w = _ce_sort(w, k, ds_, row0=si * 128)
                ds_ >>= 1
            segs.append(w)
        v = jnp.concatenate(segs, axis=0)
    return v


def _cleanup(v):
    for d in (64, 32, 16):
        v = _ce_clean(v, d)
    segs = []
    for si in range(v.shape[0] // 128):
        w = v[si * 128:(si + 1) * 128]
        for d in (8, 4, 2, 1):
            w = _ce_clean(w, d)
        segs.append(w)
    return jnp.concatenate(segs, axis=0)


def _blockrev(v):
    n = v.shape[0]
    return jnp.concatenate(
        [v[i * 8:(i + 1) * 8] for i in reversed(range(n // 8))], axis=0)


def _body(x_ref, o_ref, s_ref, *, s):
    ngroups = s // _G

    def p1(g, carry):
        base = pl.multiple_of(g * _G, _G)
        v = x_ref[0, pl.ds(base, _G), :]
        s_ref[pl.ds(base, _G), :] = _sort_group(v)
        return carry

    lax.fori_loop(0, ngroups, p1, 0)

    def merge_groups(ga, gb):
        a = s_ref[pl.ds(pl.multiple_of(ga * _G, _G), _G), :]
        b = s_ref[pl.ds(pl.multiple_of(gb * _G, _G), _G), :]
        m = _cleanup(jnp.maximum(a, _blockrev(b)))
        s_ref[pl.ds(pl.multiple_of(ga * _G, _G), _G), :] = m

    def l1(u, carry):
        merge_groups(2 * u, 2 * u + 1)
        return carry

    lax.fori_loop(0, ngroups // 2, l1, 0)
    merge_groups(0, 2)
    merge_groups(4, 6)
    merge_groups(0, 4)

    v = s_ref[0:_G, :]
    for shift in (1, 2, 4):
        w = _blockrev(v)
        w = jnp.concatenate([w[shift:], w[:shift]], axis=0)
        v = _cleanup(jnp.maximum(v, w))

    o_ref[0] = v.reshape(_K, 8, v.shape[-1])[:, 0, :]


def kernel(inputs):
    b, s, d = inputs.shape
    grid = (b, d // _LANES)
    out = pl.pallas_call(
        functools.partial(_body, s=s),
        grid=grid,
        in_specs=[pl.BlockSpec((1, s, _LANES), lambda bi, j: (bi, 0, j))],
        out_specs=pl.BlockSpec((1, _K, _LANES), lambda bi, j: (bi, 0, j)),
        out_shape=jax.ShapeDtypeStruct((b, _K, d), jnp.float32),
        scratch_shapes=[pltpu.VMEM((s, _LANES), jnp.float32)],
    )(inputs)
    return out
